# baseline (speedup 1.0000x reference)
import jax
import jax.numpy as jnp
from jax.experimental import pallas as pl
from jax.experimental.pallas import tpu as pltpu


def _pass1(x_ref, w_ref, b_ref, ct_ref, l_ref, q_ref, fq_ref):
    l = jnp.dot(x_ref[...], w_ref[...], preferred_element_type=jnp.float32)
    l = l + b_ref[...]
    l_ref[...] = l
    ct = ct_ref[...]
    csq = jnp.sum(ct * ct, axis=0, keepdims=True)
    lc = jnp.dot(l, ct, preferred_element_type=jnp.float32)
    lsq = jnp.sum(l * l, axis=1, keepdims=True)
    dist = lsq + csq - 2.0 * lc
    num = 1.0 / (1.0 + dist)
    q = num / jnp.sum(num, axis=1, keepdims=True)
    q_ref[...] = q
    fq_ref[0, 0, :] = jnp.sum(q, axis=0)


def _pass2(l_ref, ct_ref, fqp_ref, p_ref):
    l = l_ref[...]
    ct = ct_ref[...]
    csq = jnp.sum(ct * ct, axis=0, keepdims=True)
    lc = jnp.dot(l, ct, preferred_element_type=jnp.float32)
    lsq = jnp.sum(l * l, axis=1, keepdims=True)
    dist = lsq + csq - 2.0 * lc
    num = 1.0 / (1.0 + dist)
    fq = jnp.sum(fqp_ref[:, 0, :], axis=0, keepdims=True)
    r = (num * num) / fq
    p_ref[...] = r / jnp.sum(r, axis=1, keepdims=True)


def kernel(x, W, b, centroids):
    n, d_in = x.shape
    d_lat = W.shape[1]
    k = centroids.shape[0]
    bn1 = min(2048, n)
    g1 = n // bn1
    ct = centroids.T
    b2 = b.reshape(1, d_lat)

    l, q, fq_part = pl.pallas_call(
        _pass1,
        grid=(g1,),
        in_specs=[
            pl.BlockSpec((bn1, d_in), lambda i: (i, 0)),
            pl.BlockSpec((d_in, d_lat), lambda i: (0, 0)),
            pl.BlockSpec((1, d_lat), lambda i: (0, 0)),
            pl.BlockSpec((d_lat, k), lambda i: (0, 0)),
        ],
        out_specs=[
            pl.BlockSpec((bn1, d_lat), lambda i: (i, 0)),
            pl.BlockSpec((bn1, k), lambda i: (i, 0)),
            pl.BlockSpec((1, 1, k), lambda i: (i, 0, 0)),
        ],
        out_shape=[
            jax.ShapeDtypeStruct((n, d_lat), jnp.float32),
            jax.ShapeDtypeStruct((n, k), jnp.float32),
            jax.ShapeDtypeStruct((g1, 1, k), jnp.float32),
        ],
        compiler_params=pltpu.CompilerParams(
            dimension_semantics=("parallel",),
            vmem_limit_bytes=56 * 1024 * 1024,
        ),
        name="dtc_pass1",
    )(x, W, b2, ct)

    bn2 = min(4096, n)
    g2 = n // bn2
    p = pl.pallas_call(
        _pass2,
        grid=(g2,),
        in_specs=[
            pl.BlockSpec((bn2, d_lat), lambda i: (i, 0)),
            pl.BlockSpec((d_lat, k), lambda i: (0, 0)),
            pl.BlockSpec((g1, 1, k), lambda i: (0, 0, 0)),
        ],
        out_specs=pl.BlockSpec((bn2, k), lambda i: (i, 0)),
        out_shape=jax.ShapeDtypeStruct((n, k), jnp.float32),
        compiler_params=pltpu.CompilerParams(
            dimension_semantics=("parallel",),
            vmem_limit_bytes=56 * 1024 * 1024,
        ),
        name="dtc_pass2",
    )(l, ct, fq_part)

    return (l, q, p)


# speedup vs baseline: 1.0042x; 1.0042x over previous
import jax
import jax.numpy as jnp
from jax.experimental import pallas as pl
from jax.experimental.pallas import tpu as pltpu

_DN_T = (((1,), (1,)), ((), ()))


def _pass1(x_ref, w_ref, b_ref, c_ref, l_ref, q_ref, fq_ref):
    l = jnp.dot(x_ref[...], w_ref[...], preferred_element_type=jnp.float32)
    l = l + b_ref[...]
    l_ref[...] = l
    c = c_ref[...]
    ones = jnp.ones((1, c.shape[1]), jnp.float32)
    csq = jax.lax.dot_general(ones, c * c, _DN_T,
                              preferred_element_type=jnp.float32)
    lc = jax.lax.dot_general(l, c, _DN_T,
                             preferred_element_type=jnp.float32)
    lsq = jnp.sum(l * l, axis=1, keepdims=True)
    dist = lsq + csq - 2.0 * lc
    num = 1.0 / (1.0 + dist)
    q = num / jnp.sum(num, axis=1, keepdims=True)
    q_ref[...] = q
    fq_ref[0, 0, :] = jnp.sum(q, axis=0)


def _pass2(l_ref, c_ref, fqp_ref, p_ref):
    l = l_ref[...]
    c = c_ref[...]
    ones = jnp.ones((1, c.shape[1]), jnp.float32)
    csq = jax.lax.dot_general(ones, c * c, _DN_T,
                              preferred_element_type=jnp.float32)
    lc = jax.lax.dot_general(l, c, _DN_T,
                             preferred_element_type=jnp.float32)
    lsq = jnp.sum(l * l, axis=1, keepdims=True)
    dist = lsq + csq - 2.0 * lc
    num = 1.0 / (1.0 + dist)
    fq = jnp.sum(fqp_ref[:, 0, :], axis=0, keepdims=True)
    r = (num * num) / fq
    p_ref[...] = r / jnp.sum(r, axis=1, keepdims=True)


def kernel(x, W, b, centroids):
    n, d_in = x.shape
    d_lat = W.shape[1]
    k = centroids.shape[0]
    b2 = b.reshape(1, d_lat)

    bn1 = min(2048, n)
    g1 = n // bn1
    l, q, fq_part = pl.pallas_call(
        _pass1,
        grid=(g1,),
        in_specs=[
            pl.BlockSpec((bn1, d_in), lambda i: (i, 0)),
            pl.BlockSpec((d_in, d_lat), lambda i: (0, 0)),
            pl.BlockSpec((1, d_lat), lambda i: (0, 0)),
            pl.BlockSpec((k, d_lat), lambda i: (0, 0)),
        ],
        out_specs=[
            pl.BlockSpec((bn1, d_lat), lambda i: (i, 0)),
            pl.BlockSpec((bn1, k), lambda i: (i, 0)),
            pl.BlockSpec((1, 1, k), lambda i: (i, 0, 0)),
        ],
        out_shape=[
            jax.ShapeDtypeStruct((n, d_lat), jnp.float32),
            jax.ShapeDtypeStruct((n, k), jnp.float32),
            jax.ShapeDtypeStruct((g1, 1, k), jnp.float32),
        ],
        compiler_params=pltpu.CompilerParams(
            dimension_semantics=("parallel",),
            vmem_limit_bytes=56 * 1024 * 1024,
        ),
        name="dtc_pass1",
    )(x, W, b2, centroids)

    bn2 = min(4096, n)
    g2 = n // bn2
    p = pl.pallas_call(
        _pass2,
        grid=(g2,),
        in_specs=[
            pl.BlockSpec((bn2, d_lat), lambda i: (i, 0)),
            pl.BlockSpec((k, d_lat), lambda i: (0, 0)),
            pl.BlockSpec((g1, 1, k), lambda i: (0, 0, 0)),
        ],
        out_specs=pl.BlockSpec((bn2, k), lambda i: (i, 0)),
        out_shape=jax.ShapeDtypeStruct((n, k), jnp.float32),
        compiler_params=pltpu.CompilerParams(
            dimension_semantics=("parallel",),
            vmem_limit_bytes=56 * 1024 * 1024,
        ),
        name="dtc_pass2",
    )(l, centroids, fq_part)

    return (l, q, p)
